# baseline (speedup 1.0000x reference)
import functools

import jax
import jax.numpy as jnp
from jax.experimental import pallas as pl
from jax.experimental.pallas import tpu as pltpu
from jax.experimental.pallas import tpu_sc as plsc

B = 16384
F = 26
V = 100000
D = 16
TL = 30
TV = 100000
VF = 128
AF = 128
H1 = 32
H2 = 32

FD = F * D
TD = TL * D
N2 = B * F
NT = B * TL
W2 = 1024
WT = 1280

R = 1024
VC = 4096
VP = 25 * VC
FG = 4


def _sc_gather_early(fm1_flat, title_table, idx1, idxt):
    mesh = plsc.VectorSubcoreMesh(core_axis_name="c", subcore_axis_name="s")

    @functools.partial(
        pl.kernel,
        mesh=mesh,
        compiler_params=pltpu.CompilerParams(use_tc_tiling_on_sc=False),
        out_type=[
            jax.ShapeDtypeStruct((N2,), jnp.float32),
            jax.ShapeDtypeStruct((NT, D), jnp.float32),
        ],
    )
    def k(fm1_hbm, tt_hbm, idx1_hbm, idxt_hbm, g1_hbm, tg_hbm):
        def body_fm1(i1_vmem, o_vals):
            pltpu.sync_copy(fm1_hbm.at[i1_vmem.at[0]], o_vals)

        pltpu.emit_pipeline(
            body_fm1,
            grid=(N2 // W2,),
            in_specs=[pl.BlockSpec((1, W2), lambda i: (0, i))],
            out_specs=[pl.BlockSpec((W2,), lambda i: (i,))],
            core_axis_name=("c", "s"),
            dimension_semantics=(pltpu.PARALLEL,),
        )(idx1_hbm, g1_hbm)

        def body_title(i_vmem, o_rows):
            pltpu.sync_copy(tt_hbm.at[i_vmem.at[0]], o_rows)

        pltpu.emit_pipeline(
            body_title,
            grid=(NT // WT,),
            in_specs=[pl.BlockSpec((1, WT), lambda i: (0, i))],
            out_specs=[pl.BlockSpec((WT, D), lambda i: (i, 0))],
            core_axis_name=("c", "s"),
            dimension_semantics=(pltpu.PARALLEL,),
        )(idxt_hbm, tg_hbm)

    return k(fm1_flat, title_table, idx1, idxt)


def _sc_gather_fm2(fm2_flat, idx2):
    mesh = plsc.VectorSubcoreMesh(core_axis_name="c", subcore_axis_name="s")

    @functools.partial(
        pl.kernel,
        mesh=mesh,
        compiler_params=pltpu.CompilerParams(use_tc_tiling_on_sc=False),
        out_type=jax.ShapeDtypeStruct((N2, D), jnp.float32),
    )
    def k(fm2_hbm, idx2_hbm, e2_hbm):
        def body_fm(i2_vmem, o_rows):
            pltpu.sync_copy(fm2_hbm.at[i2_vmem.at[0]], o_rows)

        pltpu.emit_pipeline(
            body_fm,
            grid=(N2 // W2,),
            in_specs=[pl.BlockSpec((1, W2), lambda i: (0, i))],
            out_specs=[pl.BlockSpec((W2, D), lambda i: (i, 0))],
            core_axis_name=("c", "s"),
            dimension_semantics=(pltpu.PARALLEL,),
        )(idx2_hbm, e2_hbm)

    return k(fm2_flat, idx2)


def _tc_table_body(in_ref, p_ref, o_ref):
    x = in_ref[...]
    xc = x.reshape(8 * D, VC)
    tdn = (((0,), (0,)), ((), ()))
    o_ref[0] = jax.lax.dot_general(xc, p_ref[...], tdn,
                                   preferred_element_type=jnp.float32)


def _tc_table_transpose(fm2t3):
    place = jnp.eye(8 * D, dtype=jnp.float32)
    return pl.pallas_call(
        _tc_table_body,
        grid=(FG, VP // VC),
        in_specs=[pl.BlockSpec((8, D, VC), lambda i, j: (i, 0, j)),
                  pl.BlockSpec((8 * D, 8 * D), lambda i, j: (0, 0))],
        out_specs=pl.BlockSpec((1, VC, 8 * D), lambda i, j: (i, j, 0)),
        out_shape=jax.ShapeDtypeStruct((FG, VP, 8 * D), jnp.float32),
    )(fm2t3, place)


def _tc_fm1_body(in_ref, o_ref):
    o_ref[pl.ds(0, V)] = in_ref[0, 0, :]


def _tc_fm1_depad(fm1v):
    return pl.pallas_call(
        _tc_fm1_body,
        grid=(F,),
        in_specs=[pl.BlockSpec((1, 1, V), lambda i: (i, 0, 0))],
        out_specs=pl.BlockSpec((VP,), lambda i: (i,)),
        out_shape=jax.ShapeDtypeStruct((F * VP,), jnp.float32),
    )(fm1v)


def _tc_body(e2_ref, g1_ref, tg_ref, xv_ref, vid_ref, aud_ref,
             wv_ref, bv_ref, wa_ref, ba_ref, w1_ref, b1_ref, w2_ref, b2_ref,
             bias_ref, ex_ref, s26_ref, s30_ref, o_ref):
    f32 = jnp.float32
    e2b = e2_ref[...]
    xv = xv_ref[...]
    xvr = jnp.dot(xv, ex_ref[...], preferred_element_type=f32)
    scaled = e2b * xvr

    tp = jnp.dot(tg_ref[...], s30_ref[...], preferred_element_type=f32) * (1.0 / TL)
    vemb = jnp.dot(vid_ref[...], wv_ref[...], preferred_element_type=f32) + bv_ref[...]
    aemb = jnp.dot(aud_ref[...], wa_ref[...], preferred_element_type=f32) + ba_ref[...]

    summed = (jnp.dot(scaled, s26_ref[...], preferred_element_type=f32)
              + tp + vemb + aemb)
    sumsq = (jnp.sum(scaled * scaled, axis=1) + jnp.sum(tp * tp, axis=1)
             + jnp.sum(vemb * vemb, axis=1) + jnp.sum(aemb * aemb, axis=1))
    second_sum = 0.5 * (jnp.sum(summed * summed, axis=1) - sumsq)

    w1 = w1_ref[...]
    z = (jnp.dot(scaled, w1[0:FD, :], preferred_element_type=f32)
         + jnp.dot(tp, w1[FD:FD + D, :], preferred_element_type=f32)
         + jnp.dot(vemb, w1[FD + D:FD + 2 * D, :], preferred_element_type=f32)
         + jnp.dot(aemb, w1[FD + 2 * D:FD + 3 * D, :], preferred_element_type=f32)
         + b1_ref[...])
    h = jnp.maximum(z, 0.0)
    h = jnp.maximum(jnp.dot(h, w2_ref[...], preferred_element_type=f32) + b2_ref[...], 0.0)

    first_sum = jnp.sum(g1_ref[...] * xv, axis=1)
    tot = bias_ref[0, 0] + first_sum + second_sum + jnp.sum(h, axis=1)
    o_ref[...] = tot[:, None]


def _tc_forward(e2, g1, tg, xv, video, audio, wv, bv, wa, ba, w1, b1, w2, b2,
                bias, ex, s26, s30):
    full = lambda shape: pl.BlockSpec(shape, lambda i: tuple(0 for _ in shape))
    return pl.pallas_call(
        _tc_body,
        grid=(B // R,),
        in_specs=[
            pl.BlockSpec((R, FD), lambda i: (i, 0)),
            pl.BlockSpec((R, F), lambda i: (i, 0)),
            pl.BlockSpec((R, TD), lambda i: (i, 0)),
            pl.BlockSpec((R, F), lambda i: (i, 0)),
            pl.BlockSpec((R, VF), lambda i: (i, 0)),
            pl.BlockSpec((R, AF), lambda i: (i, 0)),
            full((VF, D)), full((1, D)),
            full((AF, D)), full((1, D)),
            full(((F + 3) * D, H1)), full((1, H1)),
            full((H1, H2)), full((1, H2)),
            full((1, 1)),
            full((F, FD)),
            full((FD, D)),
            full((TD, D)),
        ],
        out_specs=pl.BlockSpec((R, 1), lambda i: (i, 0)),
        out_shape=jax.ShapeDtypeStruct((B, 1), jnp.float32),
    )(e2, g1, tg, xv, video, audio, wv, bv, wa, ba, w1, b1, w2, b2, bias,
      ex, s26, s30)


def kernel(Xi, Xv, title, video, audio, fm_first, fm_second, title_table,
           Wv, bv, Wa, ba, W1, b1, W2, b2, bias):
    f32 = jnp.float32
    fm2t3 = jnp.transpose(fm_second, (0, 2, 1))
    fm1_flat = _tc_fm1_depad(jnp.transpose(fm_first, (0, 2, 1)))
    xi32 = Xi.astype(jnp.int32)
    farange = jnp.arange(F, dtype=jnp.int32)
    idx2 = (((farange // 8) * VP)[None, :] + xi32) * 8 + (farange % 8)[None, :]
    idx2 = idx2.reshape(1, N2)
    idx1 = (xi32 + (farange * VP)[None, :]).reshape(1, N2)
    idxt = title.astype(jnp.int32).reshape(1, NT)

    fm2t3, fm1_flat, idx1, idxt = jax.lax.optimization_barrier(
        (fm2t3, fm1_flat, idx1, idxt))
    g1, tg = _sc_gather_early(fm1_flat, title_table, idx1, idxt)
    fm2_flat = _tc_table_transpose(fm2t3).reshape(FG * VP * 8, D)
    e2 = _sc_gather_fm2(fm2_flat, idx2)

    jf = jnp.arange(FD)
    ex = (jf[None, :] // D == jnp.arange(F)[:, None]).astype(f32)
    s26 = (jf[:, None] % D == jnp.arange(D)[None, :]).astype(f32)
    jt = jnp.arange(TD)
    s30 = (jt[:, None] % D == jnp.arange(D)[None, :]).astype(f32)

    out = _tc_forward(
        e2.reshape(B, FD), g1.reshape(B, F), tg.reshape(B, TD),
        Xv, video, audio, Wv, bv.reshape(1, D), Wa, ba.reshape(1, D),
        W1, b1.reshape(1, H1), W2, b2.reshape(1, H2),
        bias.reshape(1, 1).astype(f32), ex, s26, s30)
    return out[:, 0]


# speedup vs baseline: 1.0931x; 1.0931x over previous
import functools

import jax
import jax.numpy as jnp
from jax.experimental import pallas as pl
from jax.experimental.pallas import tpu as pltpu
from jax.experimental.pallas import tpu_sc as plsc

B = 16384
F = 26
V = 100000
D = 16
TL = 30
TV = 100000
VF = 128
AF = 128
H1 = 32
H2 = 32

FD = F * D
TD = TL * D
N2 = B * F
NT = B * TL
W2 = 1024
WT = 1280

R = 1024
VC = 12800
VP = 102400
FG = 4


def _sc_gather_early(fm1_flat, title_table, idx1, idxt):
    mesh = plsc.VectorSubcoreMesh(core_axis_name="c", subcore_axis_name="s")

    @functools.partial(
        pl.kernel,
        mesh=mesh,
        compiler_params=pltpu.CompilerParams(use_tc_tiling_on_sc=False),
        out_type=[
            jax.ShapeDtypeStruct((N2,), jnp.float32),
            jax.ShapeDtypeStruct((NT, D), jnp.float32),
        ],
    )
    def k(fm1_hbm, tt_hbm, idx1_hbm, idxt_hbm, g1_hbm, tg_hbm):
        def body_fm1(i1_vmem, o_vals):
            pltpu.sync_copy(fm1_hbm.at[i1_vmem.at[0]], o_vals)

        pltpu.emit_pipeline(
            body_fm1,
            grid=(N2 // W2,),
            in_specs=[pl.BlockSpec((1, W2), lambda i: (0, i))],
            out_specs=[pl.BlockSpec((W2,), lambda i: (i,))],
            core_axis_name=("c", "s"),
            dimension_semantics=(pltpu.PARALLEL,),
        )(idx1_hbm, g1_hbm)

        def body_title(i_vmem, o_rows):
            pltpu.sync_copy(tt_hbm.at[i_vmem.at[0]], o_rows)

        pltpu.emit_pipeline(
            body_title,
            grid=(NT // WT,),
            in_specs=[pl.BlockSpec((1, WT), lambda i: (0, i))],
            out_specs=[pl.BlockSpec((WT, D), lambda i: (i, 0))],
            core_axis_name=("c", "s"),
            dimension_semantics=(pltpu.PARALLEL,),
        )(idxt_hbm, tg_hbm)

    return k(fm1_flat, title_table, idx1, idxt)


def _sc_gather_fm2(fm2_flat, idx2):
    mesh = plsc.VectorSubcoreMesh(core_axis_name="c", subcore_axis_name="s")

    @functools.partial(
        pl.kernel,
        mesh=mesh,
        compiler_params=pltpu.CompilerParams(use_tc_tiling_on_sc=False),
        out_type=jax.ShapeDtypeStruct((N2, D), jnp.float32),
    )
    def k(fm2_hbm, idx2_hbm, e2_hbm):
        def body_fm(i2_vmem, o_rows):
            pltpu.sync_copy(fm2_hbm.at[i2_vmem.at[0]], o_rows)

        pltpu.emit_pipeline(
            body_fm,
            grid=(N2 // W2,),
            in_specs=[pl.BlockSpec((1, W2), lambda i: (0, i))],
            out_specs=[pl.BlockSpec((W2, D), lambda i: (i, 0))],
            core_axis_name=("c", "s"),
            dimension_semantics=(pltpu.PARALLEL,),
        )(idx2_hbm, e2_hbm)

    return k(fm2_flat, idx2)


def _tc_table_body(in_ref, p_ref, o_ref):
    x = in_ref[...]
    xc = x.reshape(8 * D, VC)
    tdn = (((0,), (0,)), ((), ()))
    o_ref[0] = jax.lax.dot_general(xc, p_ref[...], tdn,
                                   preferred_element_type=jnp.float32)


def _tc_table_transpose(fm2t3):
    place = jnp.eye(8 * D, dtype=jnp.float32)
    return pl.pallas_call(
        _tc_table_body,
        grid=(FG, VP // VC),
        in_specs=[pl.BlockSpec((8, D, VC), lambda i, j: (i, 0, j)),
                  pl.BlockSpec((8 * D, 8 * D), lambda i, j: (0, 0))],
        out_specs=pl.BlockSpec((1, VC, 8 * D), lambda i, j: (i, j, 0)),
        out_shape=jax.ShapeDtypeStruct((FG, VP, 8 * D), jnp.float32),
    )(fm2t3, place)


def _tc_fm1_body(in_ref, o_ref):
    o_ref[pl.ds(0, V)] = in_ref[0, 0, :]


def _tc_fm1_depad(fm1v):
    return pl.pallas_call(
        _tc_fm1_body,
        grid=(F,),
        in_specs=[pl.BlockSpec((1, 1, V), lambda i: (i, 0, 0))],
        out_specs=pl.BlockSpec((VP,), lambda i: (i,)),
        out_shape=jax.ShapeDtypeStruct((F * VP,), jnp.float32),
    )(fm1v)


def _tc_body(e2_ref, g1_ref, tg_ref, xv_ref, vid_ref, aud_ref,
             wv_ref, bv_ref, wa_ref, ba_ref, w1_ref, b1_ref, w2_ref, b2_ref,
             bias_ref, ex_ref, s26_ref, s30_ref, o_ref):
    f32 = jnp.float32
    e2b = e2_ref[...]
    xv = xv_ref[...]
    xvr = jnp.dot(xv, ex_ref[...], preferred_element_type=f32)
    scaled = e2b * xvr

    tp = jnp.dot(tg_ref[...], s30_ref[...], preferred_element_type=f32) * (1.0 / TL)
    vemb = jnp.dot(vid_ref[...], wv_ref[...], preferred_element_type=f32) + bv_ref[...]
    aemb = jnp.dot(aud_ref[...], wa_ref[...], preferred_element_type=f32) + ba_ref[...]

    summed = (jnp.dot(scaled, s26_ref[...], preferred_element_type=f32)
              + tp + vemb + aemb)
    sumsq = (jnp.sum(scaled * scaled, axis=1) + jnp.sum(tp * tp, axis=1)
             + jnp.sum(vemb * vemb, axis=1) + jnp.sum(aemb * aemb, axis=1))
    second_sum = 0.5 * (jnp.sum(summed * summed, axis=1) - sumsq)

    w1 = w1_ref[...]
    z = (jnp.dot(scaled, w1[0:FD, :], preferred_element_type=f32)
         + jnp.dot(tp, w1[FD:FD + D, :], preferred_element_type=f32)
         + jnp.dot(vemb, w1[FD + D:FD + 2 * D, :], preferred_element_type=f32)
         + jnp.dot(aemb, w1[FD + 2 * D:FD + 3 * D, :], preferred_element_type=f32)
         + b1_ref[...])
    h = jnp.maximum(z, 0.0)
    h = jnp.maximum(jnp.dot(h, w2_ref[...], preferred_element_type=f32) + b2_ref[...], 0.0)

    first_sum = jnp.sum(g1_ref[...] * xv, axis=1)
    tot = bias_ref[0, 0] + first_sum + second_sum + jnp.sum(h, axis=1)
    o_ref[...] = tot[:, None]


def _tc_forward(e2, g1, tg, xv, video, audio, wv, bv, wa, ba, w1, b1, w2, b2,
                bias, ex, s26, s30):
    full = lambda shape: pl.BlockSpec(shape, lambda i: tuple(0 for _ in shape))
    return pl.pallas_call(
        _tc_body,
        grid=(B // R,),
        in_specs=[
            pl.BlockSpec((R, FD), lambda i: (i, 0)),
            pl.BlockSpec((R, F), lambda i: (i, 0)),
            pl.BlockSpec((R, TD), lambda i: (i, 0)),
            pl.BlockSpec((R, F), lambda i: (i, 0)),
            pl.BlockSpec((R, VF), lambda i: (i, 0)),
            pl.BlockSpec((R, AF), lambda i: (i, 0)),
            full((VF, D)), full((1, D)),
            full((AF, D)), full((1, D)),
            full(((F + 3) * D, H1)), full((1, H1)),
            full((H1, H2)), full((1, H2)),
            full((1, 1)),
            full((F, FD)),
            full((FD, D)),
            full((TD, D)),
        ],
        out_specs=pl.BlockSpec((R, 1), lambda i: (i, 0)),
        out_shape=jax.ShapeDtypeStruct((B, 1), jnp.float32),
    )(e2, g1, tg, xv, video, audio, wv, bv, wa, ba, w1, b1, w2, b2, bias,
      ex, s26, s30)


def kernel(Xi, Xv, title, video, audio, fm_first, fm_second, title_table,
           Wv, bv, Wa, ba, W1, b1, W2, b2, bias):
    f32 = jnp.float32
    fm2t3 = jnp.transpose(fm_second, (0, 2, 1))
    fm1_flat = _tc_fm1_depad(jnp.transpose(fm_first, (0, 2, 1)))
    xi32 = Xi.astype(jnp.int32)
    farange = jnp.arange(F, dtype=jnp.int32)
    idx2 = (((farange // 8) * VP)[None, :] + xi32) * 8 + (farange % 8)[None, :]
    idx2 = idx2.reshape(1, N2)
    idx1 = (xi32 + (farange * VP)[None, :]).reshape(1, N2)
    idxt = title.astype(jnp.int32).reshape(1, NT)

    g1, tg = _sc_gather_early(fm1_flat, title_table, idx1, idxt)
    fm2_flat = _tc_table_transpose(fm2t3).reshape(FG * VP * 8, D)
    e2 = _sc_gather_fm2(fm2_flat, idx2)

    jf = jnp.arange(FD)
    ex = (jf[None, :] // D == jnp.arange(F)[:, None]).astype(f32)
    s26 = (jf[:, None] % D == jnp.arange(D)[None, :]).astype(f32)
    jt = jnp.arange(TD)
    s30 = (jt[:, None] % D == jnp.arange(D)[None, :]).astype(f32)

    out = _tc_forward(
        e2.reshape(B, FD), g1.reshape(B, F), tg.reshape(B, TD),
        Xv, video, audio, Wv, bv.reshape(1, D), Wa, ba.reshape(1, D),
        W1, b1.reshape(1, H1), W2, b2.reshape(1, H2),
        bias.reshape(1, 1).astype(f32), ex, s26, s30)
    return out[:, 0]


# speedup vs baseline: 1.0978x; 1.0044x over previous
import functools

import jax
import jax.numpy as jnp
from jax.experimental import pallas as pl
from jax.experimental.pallas import tpu as pltpu
from jax.experimental.pallas import tpu_sc as plsc

B = 16384
F = 26
V = 100000
D = 16
TL = 30
TV = 100000
VF = 128
AF = 128
H1 = 32
H2 = 32

FD = F * D
TD = TL * D
N2 = B * F
NT = B * TL
W2 = 2048
WT = 2560

R = 1024
VC = 12800
VP = 102400
FG = 4


def _sc_gather_early(fm1_flat, title_table, idx1, idxt):
    mesh = plsc.VectorSubcoreMesh(core_axis_name="c", subcore_axis_name="s")

    @functools.partial(
        pl.kernel,
        mesh=mesh,
        compiler_params=pltpu.CompilerParams(use_tc_tiling_on_sc=False),
        out_type=[
            jax.ShapeDtypeStruct((N2,), jnp.float32),
            jax.ShapeDtypeStruct((NT, D), jnp.float32),
        ],
    )
    def k(fm1_hbm, tt_hbm, idx1_hbm, idxt_hbm, g1_hbm, tg_hbm):
        def body_fm1(i1_vmem, o_vals):
            pltpu.sync_copy(fm1_hbm.at[i1_vmem.at[0]], o_vals)

        pltpu.emit_pipeline(
            body_fm1,
            grid=(N2 // W2,),
            in_specs=[pl.BlockSpec((1, W2), lambda i: (0, i))],
            out_specs=[pl.BlockSpec((W2,), lambda i: (i,))],
            core_axis_name=("c", "s"),
            dimension_semantics=(pltpu.PARALLEL,),
        )(idx1_hbm, g1_hbm)

        def body_title(i_vmem, o_rows):
            pltpu.sync_copy(tt_hbm.at[i_vmem.at[0]], o_rows)

        pltpu.emit_pipeline(
            body_title,
            grid=(NT // WT,),
            in_specs=[pl.BlockSpec((1, WT), lambda i: (0, i))],
            out_specs=[pl.BlockSpec((WT, D), lambda i: (i, 0))],
            core_axis_name=("c", "s"),
            dimension_semantics=(pltpu.PARALLEL,),
        )(idxt_hbm, tg_hbm)

    return k(fm1_flat, title_table, idx1, idxt)


def _sc_gather_fm2(fm2_flat, idx2):
    mesh = plsc.VectorSubcoreMesh(core_axis_name="c", subcore_axis_name="s")

    @functools.partial(
        pl.kernel,
        mesh=mesh,
        compiler_params=pltpu.CompilerParams(use_tc_tiling_on_sc=False),
        out_type=jax.ShapeDtypeStruct((N2, D), jnp.float32),
    )
    def k(fm2_hbm, idx2_hbm, e2_hbm):
        def body_fm(i2_vmem, o_rows):
            pltpu.sync_copy(fm2_hbm.at[i2_vmem.at[0]], o_rows)

        pltpu.emit_pipeline(
            body_fm,
            grid=(N2 // W2,),
            in_specs=[pl.BlockSpec((1, W2), lambda i: (0, i))],
            out_specs=[pl.BlockSpec((W2, D), lambda i: (i, 0))],
            core_axis_name=("c", "s"),
            dimension_semantics=(pltpu.PARALLEL,),
        )(idx2_hbm, e2_hbm)

    return k(fm2_flat, idx2)


def _tc_table_body(in_ref, p_ref, o_ref):
    x = in_ref[...]
    xc = x.reshape(8 * D, VC)
    tdn = (((0,), (0,)), ((), ()))
    o_ref[0] = jax.lax.dot_general(xc, p_ref[...], tdn,
                                   preferred_element_type=jnp.float32)


def _tc_table_transpose(fm2t3):
    place = jnp.eye(8 * D, dtype=jnp.float32)
    return pl.pallas_call(
        _tc_table_body,
        grid=(FG, VP // VC),
        in_specs=[pl.BlockSpec((8, D, VC), lambda i, j: (i, 0, j)),
                  pl.BlockSpec((8 * D, 8 * D), lambda i, j: (0, 0))],
        out_specs=pl.BlockSpec((1, VC, 8 * D), lambda i, j: (i, j, 0)),
        out_shape=jax.ShapeDtypeStruct((FG, VP, 8 * D), jnp.float32),
    )(fm2t3, place)


def _tc_fm1_body(in_ref, o_ref):
    o_ref[pl.ds(0, V)] = in_ref[0, 0, :]


def _tc_fm1_depad(fm1v):
    return pl.pallas_call(
        _tc_fm1_body,
        grid=(F,),
        in_specs=[pl.BlockSpec((1, 1, V), lambda i: (i, 0, 0))],
        out_specs=pl.BlockSpec((VP,), lambda i: (i,)),
        out_shape=jax.ShapeDtypeStruct((F * VP,), jnp.float32),
    )(fm1v)


def _tc_body(e2_ref, g1_ref, tg_ref, xv_ref, vid_ref, aud_ref,
             wv_ref, bv_ref, wa_ref, ba_ref, w1_ref, b1_ref, w2_ref, b2_ref,
             bias_ref, ex_ref, s26_ref, s30_ref, o_ref):
    f32 = jnp.float32
    e2b = e2_ref[...]
    xv = xv_ref[...]
    xvr = jnp.dot(xv, ex_ref[...], preferred_element_type=f32)
    scaled = e2b * xvr

    tp = jnp.dot(tg_ref[...], s30_ref[...], preferred_element_type=f32) * (1.0 / TL)
    vemb = jnp.dot(vid_ref[...], wv_ref[...], preferred_element_type=f32) + bv_ref[...]
    aemb = jnp.dot(aud_ref[...], wa_ref[...], preferred_element_type=f32) + ba_ref[...]

    summed = (jnp.dot(scaled, s26_ref[...], preferred_element_type=f32)
              + tp + vemb + aemb)
    sumsq = (jnp.sum(scaled * scaled, axis=1) + jnp.sum(tp * tp, axis=1)
             + jnp.sum(vemb * vemb, axis=1) + jnp.sum(aemb * aemb, axis=1))
    second_sum = 0.5 * (jnp.sum(summed * summed, axis=1) - sumsq)

    w1 = w1_ref[...]
    z = (jnp.dot(scaled, w1[0:FD, :], preferred_element_type=f32)
         + jnp.dot(tp, w1[FD:FD + D, :], preferred_element_type=f32)
         + jnp.dot(vemb, w1[FD + D:FD + 2 * D, :], preferred_element_type=f32)
         + jnp.dot(aemb, w1[FD + 2 * D:FD + 3 * D, :], preferred_element_type=f32)
         + b1_ref[...])
    h = jnp.maximum(z, 0.0)
    h = jnp.maximum(jnp.dot(h, w2_ref[...], preferred_element_type=f32) + b2_ref[...], 0.0)

    first_sum = jnp.sum(g1_ref[...] * xv, axis=1)
    tot = bias_ref[0, 0] + first_sum + second_sum + jnp.sum(h, axis=1)
    o_ref[...] = tot[:, None]


def _tc_forward(e2, g1, tg, xv, video, audio, wv, bv, wa, ba, w1, b1, w2, b2,
                bias, ex, s26, s30):
    full = lambda shape: pl.BlockSpec(shape, lambda i: tuple(0 for _ in shape))
    return pl.pallas_call(
        _tc_body,
        grid=(B // R,),
        in_specs=[
            pl.BlockSpec((R, FD), lambda i: (i, 0)),
            pl.BlockSpec((R, F), lambda i: (i, 0)),
            pl.BlockSpec((R, TD), lambda i: (i, 0)),
            pl.BlockSpec((R, F), lambda i: (i, 0)),
            pl.BlockSpec((R, VF), lambda i: (i, 0)),
            pl.BlockSpec((R, AF), lambda i: (i, 0)),
            full((VF, D)), full((1, D)),
            full((AF, D)), full((1, D)),
            full(((F + 3) * D, H1)), full((1, H1)),
            full((H1, H2)), full((1, H2)),
            full((1, 1)),
            full((F, FD)),
            full((FD, D)),
            full((TD, D)),
        ],
        out_specs=pl.BlockSpec((R, 1), lambda i: (i, 0)),
        out_shape=jax.ShapeDtypeStruct((B, 1), jnp.float32),
    )(e2, g1, tg, xv, video, audio, wv, bv, wa, ba, w1, b1, w2, b2, bias,
      ex, s26, s30)


def kernel(Xi, Xv, title, video, audio, fm_first, fm_second, title_table,
           Wv, bv, Wa, ba, W1, b1, W2, b2, bias):
    f32 = jnp.float32
    fm2t3 = jnp.transpose(fm_second, (0, 2, 1))
    fm1_flat = _tc_fm1_depad(jnp.transpose(fm_first, (0, 2, 1)))
    xi32 = Xi.astype(jnp.int32)
    farange = jnp.arange(F, dtype=jnp.int32)
    idx2 = (((farange // 8) * VP)[None, :] + xi32) * 8 + (farange % 8)[None, :]
    idx2 = idx2.reshape(1, N2)
    idx1 = (xi32 + (farange * VP)[None, :]).reshape(1, N2)
    idxt = title.astype(jnp.int32).reshape(1, NT)

    g1, tg = _sc_gather_early(fm1_flat, title_table, idx1, idxt)
    fm2_flat = _tc_table_transpose(fm2t3).reshape(FG * VP * 8, D)
    e2 = _sc_gather_fm2(fm2_flat, idx2)

    jf = jnp.arange(FD)
    ex = (jf[None, :] // D == jnp.arange(F)[:, None]).astype(f32)
    s26 = (jf[:, None] % D == jnp.arange(D)[None, :]).astype(f32)
    jt = jnp.arange(TD)
    s30 = (jt[:, None] % D == jnp.arange(D)[None, :]).astype(f32)

    out = _tc_forward(
        e2.reshape(B, FD), g1.reshape(B, F), tg.reshape(B, TD),
        Xv, video, audio, Wv, bv.reshape(1, D), Wa, ba.reshape(1, D),
        W1, b1.reshape(1, H1), W2, b2.reshape(1, H2),
        bias.reshape(1, 1).astype(f32), ex, s26, s30)
    return out[:, 0]
